# Initial kernel scaffold; baseline (speedup 1.0000x reference)
import jax
import jax.numpy as jnp
from jax.experimental import pallas as pl


def kernel(Q, K, V):
    raise NotImplementedError("write your pallas kernel here")


# speedup vs baseline: 37.5780x; 37.5780x over previous
import functools

import jax
import jax.numpy as jnp
from jax import lax
from jax.experimental import pallas as pl
from jax.experimental.pallas import tpu as pltpu

TOP_K = 64
TQ = 256


def _attn_block(q_ref, k_ref, v_ref, o_ref):
    q = q_ref[0]
    k = k_ref[0]
    v = v_ref[0]

    s = lax.dot_general(q, k, (((1,), (1,)), ((), ())),
                        preferred_element_type=jnp.float32)

    i = lax.bitcast_convert_type(s, jnp.int32)
    key = jnp.where(i >= 0, i, i ^ jnp.int32(0x7FFFFFFF))

    def count_ge(thresh):
        return jnp.sum((key >= thresh).astype(jnp.int32), axis=1, keepdims=True)

    zero = jnp.zeros((q.shape[0], 1), jnp.int32)
    t = jnp.where(count_ge(zero) >= TOP_K, 0, jnp.int32(-2147483648))
    for bit in range(30, -1, -1):
        cand = t + jnp.int32(1 << bit)
        t = jnp.where(count_ge(cand) >= TOP_K, cand, t)

    sel = key >= t
    m = jnp.max(s, axis=1, keepdims=True)
    e = jnp.where(sel, jnp.exp(s - m), 0.0)
    z = jnp.sum(e, axis=1, keepdims=True)
    p = e * (1.0 / z)

    o_ref[0] = lax.dot_general(p, v, (((1,), (0,)), ((), ())),
                               preferred_element_type=jnp.float32)


@jax.jit
def kernel(Q, K, V):
    B, Lq, d = Q.shape
    Lk = K.shape[1]
    grid = (B, Lq // TQ)
    return pl.pallas_call(
        _attn_block,
        grid=grid,
        in_specs=[
            pl.BlockSpec((1, TQ, d), lambda b, i: (b, i, 0)),
            pl.BlockSpec((1, Lk, d), lambda b, i: (b, 0, 0)),
            pl.BlockSpec((1, Lk, d), lambda b, i: (b, 0, 0)),
        ],
        out_specs=pl.BlockSpec((1, TQ, d), lambda b, i: (b, i, 0)),
        out_shape=jax.ShapeDtypeStruct((B, Lq, d), jnp.float32),
    )(Q, K, V)
